# Initial kernel scaffold; baseline (speedup 1.0000x reference)
import jax
import jax.numpy as jnp
from jax.experimental import pallas as pl


def kernel(positions, hidden_states, w_qkv, w_o, gate_w, w_gate_up, w_down, pre_attn_norm_w, post_attn_norm_w, pre_moe_norm_w, post_moe_norm_w):
    raise NotImplementedError("write your pallas kernel here")


# speedup vs baseline: 1.4306x; 1.4306x over previous
import functools

import jax
import jax.numpy as jnp
from jax import lax
from jax.experimental import pallas as pl
from jax.experimental.pallas import tpu as pltpu
from jax.experimental.pallas import tpu_sc as plsc

B, S, H = 1, 2048, 2048
NH, NKV, HD = 16, 8, 128
E, TOPK, FF = 8, 2, 2048
EPS = 1e-5
ATTN_CAP = 30.0
ROUTER_CAP = 30.0
THETA = 10000.0
ATTN_MULT = 1.0

TM = 128
G = S * TOPK // TM + E
XS = G * TM
FN = 512
RT = 256


def _rms(x, w):
    v = jnp.mean(jnp.square(x), axis=-1, keepdims=True)
    return (x * lax.rsqrt(v + EPS)) * w


def _qkv_body(hs_ref, nw_ref, w_ref, cos_ref, sin_ref, o_ref):
    n = pl.program_id(0)
    xn = _rms(hs_ref[...], nw_ref[...])
    y = jnp.dot(xn, w_ref[...], preferred_element_type=jnp.float32)
    cos = cos_ref[...]
    sin = sin_ref[...]
    parts = []
    for h in range(8):
        x1 = y[:, h * HD : h * HD + HD // 2]
        x2 = y[:, h * HD + HD // 2 : (h + 1) * HD]
        rot = jnp.concatenate([-x2, x1], axis=1)
        parts.append(y[:, h * HD : (h + 1) * HD] * cos + rot * sin)
    roped = jnp.concatenate(parts, axis=1)
    o_ref[...] = jnp.where(n < 3, roped, y)


def _qkv(hs, nw, w_qkv, cosf, sinf):
    return pl.pallas_call(
        _qkv_body,
        grid=(4, S // RT),
        in_specs=[
            pl.BlockSpec((RT, H), lambda n, r: (r, 0)),
            pl.BlockSpec((1, H), lambda n, r: (0, 0)),
            pl.BlockSpec((H, 1024), lambda n, r: (0, n)),
            pl.BlockSpec((RT, HD), lambda n, r: (r, 0)),
            pl.BlockSpec((RT, HD), lambda n, r: (r, 0)),
        ],
        out_specs=pl.BlockSpec((RT, 1024), lambda n, r: (r, n)),
        out_shape=jax.ShapeDtypeStruct((S, (NH + 2 * NKV) * HD), jnp.float32),
    )(hs, nw, w_qkv, cosf, sinf)


def _attn_body(q_ref, k_ref, v_ref, o_ref):
    qb = pl.program_id(1)
    q = q_ref[...]
    k = k_ref[...]
    s = lax.dot_general(q, k, (((1,), (1,)), ((), ())),
                        preferred_element_type=jnp.float32) * (HD ** -0.5)
    s = ATTN_CAP * jnp.tanh(s * (1.0 / ATTN_CAP))
    rows = qb * RT + lax.broadcasted_iota(jnp.int32, (RT, S), 0)
    cols = lax.broadcasted_iota(jnp.int32, (RT, S), 1)
    s = jnp.where(cols <= rows, s, -1e9)
    m = jnp.max(s, axis=1, keepdims=True)
    e = jnp.exp(s - m)
    p = e / jnp.sum(e, axis=1, keepdims=True)
    o_ref[...] = jnp.dot(p, v_ref[...], preferred_element_type=jnp.float32)


def _attn(q, k, v):
    return pl.pallas_call(
        _attn_body,
        grid=(NH, S // RT),
        in_specs=[
            pl.BlockSpec((RT, HD), lambda h, qb: (qb, h)),
            pl.BlockSpec((S, HD), lambda h, qb: (0, h // 2)),
            pl.BlockSpec((S, HD), lambda h, qb: (0, h // 2)),
        ],
        out_specs=pl.BlockSpec((RT, HD), lambda h, qb: (qb, h)),
        out_shape=jax.ShapeDtypeStruct((S, NH * HD), jnp.float32),
    )(q, k, v)


def _oproj_body(ao_ref, wo_ref, hs_ref, qa_ref, pm_ref, gw_ref,
                hid_ref, xn_ref, lg_ref):
    ao = jnp.dot(ao_ref[...], wo_ref[...], preferred_element_type=jnp.float32)
    ao = _rms(ao * ATTN_MULT, qa_ref[...])
    hid = hs_ref[...] + ao
    hid_ref[...] = hid
    xn = _rms(hid, pm_ref[...])
    xn_ref[...] = xn
    lg_ref[...] = jnp.dot(xn, gw_ref[...], preferred_element_type=jnp.float32)


def _oproj(ao, w_o, hs, qa, pm, gate_w):
    return pl.pallas_call(
        _oproj_body,
        grid=(S // RT,),
        in_specs=[
            pl.BlockSpec((RT, NH * HD), lambda r: (r, 0)),
            pl.BlockSpec((NH * HD, H), lambda r: (0, 0)),
            pl.BlockSpec((RT, H), lambda r: (r, 0)),
            pl.BlockSpec((1, H), lambda r: (0, 0)),
            pl.BlockSpec((1, H), lambda r: (0, 0)),
            pl.BlockSpec((H, E), lambda r: (0, 0)),
        ],
        out_specs=[
            pl.BlockSpec((RT, H), lambda r: (r, 0)),
            pl.BlockSpec((RT, H), lambda r: (r, 0)),
            pl.BlockSpec((RT, E), lambda r: (r, 0)),
        ],
        out_shape=[
            jax.ShapeDtypeStruct((S, H), jnp.float32),
            jax.ShapeDtypeStruct((S, H), jnp.float32),
            jax.ShapeDtypeStruct((S, E), jnp.float32),
        ],
    )(ao, w_o, hs, qa, pm, gate_w)


def _route_body(lg_ref, p0_ref, p1_ref, w0_ref, w1_ref, te_ref, tot_ref):
    T = S
    A = T * TOPK
    NB = A // TM

    l = lg_ref[...]
    l = ROUTER_CAP * jnp.tanh(l * (1.0 / ROUTER_CAP))
    mx = jnp.max(l, axis=1, keepdims=True)
    ex = jnp.exp(l - mx)
    p = ex / jnp.sum(ex, axis=1, keepdims=True)

    ei = lax.broadcasted_iota(jnp.int32, (T, E), 1)
    v1 = jnp.max(p, axis=1, keepdims=True)
    i1 = jnp.min(jnp.where(p == v1, ei, E), axis=1, keepdims=True)
    oh1 = (ei == i1).astype(jnp.float32)
    p2 = jnp.where(ei == i1, -1.0, p)
    v2 = jnp.max(p2, axis=1, keepdims=True)
    i2 = jnp.min(jnp.where(p2 == v2, ei, E), axis=1, keepdims=True)
    oh2 = (ei == i2).astype(jnp.float32)
    denom = v1 + v2
    w0_ref[...] = v1 / denom
    w1_ref[...] = v2 / denom

    M = jnp.concatenate([oh1, oh2], axis=0)

    ri = lax.broadcasted_iota(jnp.int32, (TM, TM), 0)
    ci = lax.broadcasted_iota(jnp.int32, (TM, TM), 1)
    Linc = (ri >= ci).astype(jnp.float32)
    hp = lax.Precision.HIGHEST
    Cb = [jnp.dot(Linc, M[b * TM : (b + 1) * TM, :], precision=hp,
                  preferred_element_type=jnp.float32) for b in range(NB)]
    C = jnp.concatenate(Cb, axis=0)
    Ssum = jnp.concatenate([c[TM - 1 : TM, :] for c in Cb], axis=0)
    rb = lax.broadcasted_iota(jnp.int32, (NB, NB), 0)
    cb = lax.broadcasted_iota(jnp.int32, (NB, NB), 1)
    Lstr = (rb > cb).astype(jnp.float32)
    P = jnp.dot(Lstr, Ssum, precision=hp, preferred_element_type=jnp.float32)
    blk = lax.broadcasted_iota(jnp.int32, (A, 1), 0) // TM
    Rep = (blk == lax.broadcasted_iota(jnp.int32, (A, NB), 1)).astype(jnp.float32)
    Pbig = jnp.dot(Rep, P, precision=hp, preferred_element_type=jnp.float32)
    rank = jnp.sum((C + Pbig - 1.0) * M, axis=1, keepdims=True)

    counts = Ssum[NB - 1 : NB, :] + P[NB - 1 : NB, :]
    ci32 = counts.astype(jnp.int32)
    tiles = (ci32 + (TM - 1)) >> 7
    re8 = lax.broadcasted_iota(jnp.int32, (E, E), 0)
    ce8 = lax.broadcasted_iota(jnp.int32, (E, E), 1)
    U8 = (re8 < ce8).astype(jnp.float32)
    toff = jnp.dot(tiles.astype(jnp.float32), U8, precision=hp,
                   preferred_element_type=jnp.float32)
    padded_off = toff * float(TM)
    offa = jnp.sum(M * padded_off, axis=1, keepdims=True)
    pos = (rank + offa).astype(jnp.int32)
    p0_ref[...] = pos[:T]
    p1_ref[...] = pos[T:]

    ti = lax.broadcasted_iota(jnp.int32, (TM, E), 0)
    cnt = jnp.sum((ti >= toff.astype(jnp.int32)).astype(jnp.int32),
                  axis=1, keepdims=True)
    te_ref[...] = jnp.maximum(cnt - 1, 0)
    tot_ref[...] = jnp.sum(tiles, axis=1, keepdims=True)


def _route(logits):
    return pl.pallas_call(
        _route_body,
        out_shape=[
            jax.ShapeDtypeStruct((S, 1), jnp.int32),
            jax.ShapeDtypeStruct((S, 1), jnp.int32),
            jax.ShapeDtypeStruct((S, 1), jnp.float32),
            jax.ShapeDtypeStruct((S, 1), jnp.float32),
            jax.ShapeDtypeStruct((TM, 1), jnp.int32),
            jax.ShapeDtypeStruct((1, 1), jnp.int32),
        ],
    )(logits)


def _sc_info():
    info = plsc.get_sparse_core_info()
    return info.num_cores, info.num_subcores


def _dispatch_sc(xn, p0, p1):
    NC, NS = _sc_info()
    NW = NC * NS
    tpw = S // NW
    CH = 16
    mesh = plsc.VectorSubcoreMesh(core_axis_name="c", subcore_axis_name="s")

    @functools.partial(
        pl.kernel, mesh=mesh,
        out_type=jax.ShapeDtypeStruct((XS, H), jnp.float32),
        scratch_types=[
            pltpu.VMEM((CH,), jnp.int32),
            pltpu.VMEM((CH,), jnp.int32),
            pltpu.VMEM((CH, H), jnp.float32),
            pltpu.SemaphoreType.DMA,
        ],
    )
    def k(x_hbm, p0_hbm, p1_hbm, xs_hbm, i0_v, i1_v, rows_v, sem):
        wid = lax.axis_index("s") * NC + lax.axis_index("c")
        base0 = wid * tpw
        for c in range(tpw // CH):
            base = base0 + c * CH
            pltpu.sync_copy(p0_hbm.at[pl.ds(base, CH)], i0_v)
            pltpu.sync_copy(p1_hbm.at[pl.ds(base, CH)], i1_v)
            pltpu.sync_copy(x_hbm.at[pl.ds(base, CH)], rows_v)
            pltpu.async_copy(rows_v, xs_hbm.at[i0_v], sem).wait()
            pltpu.async_copy(rows_v, xs_hbm.at[i1_v], sem).wait()

    return k(xn, p0, p1)


def _combine_sc(ys, p0, p1):
    NC, NS = _sc_info()
    NW = NC * NS
    tpw = S // NW
    CH = 16
    mesh = plsc.VectorSubcoreMesh(core_axis_name="c", subcore_axis_name="s")

    @functools.partial(
        pl.kernel, mesh=mesh,
        out_type=jax.ShapeDtypeStruct((TOPK * S, H), jnp.float32),
        scratch_types=[
            pltpu.VMEM((CH,), jnp.int32),
            pltpu.VMEM((CH, H), jnp.float32),
            pltpu.SemaphoreType.DMA,
        ],
    )
    def k(ys_hbm, p0_hbm, p1_hbm, yp_hbm, idx_v, rows_v, sem):
        wid = lax.axis_index("s") * NC + lax.axis_index("c")
        base0 = wid * tpw
        for kk, p_hbm in ((0, p0_hbm), (1, p1_hbm)):
            for c in range(tpw // CH):
                base = base0 + c * CH
                pltpu.sync_copy(p_hbm.at[pl.ds(base, CH)], idx_v)
                pltpu.async_copy(ys_hbm.at[idx_v], rows_v, sem).wait()
                pltpu.sync_copy(rows_v, yp_hbm.at[pl.ds(kk * S + base, CH)])

    return k(ys, p0, p1)


def _gelu(x):
    return 0.5 * x * (1.0 + lax.erf(x * 0.7071067811865475))


def _gmm1_body(te_ref, tot_ref, xs_ref, wg_ref, wu_ref, o_ref):
    m = pl.program_id(1)

    @pl.when(m < tot_ref[0])
    def _():
        x = xs_ref[...]
        g = jnp.dot(x, wg_ref[0], preferred_element_type=jnp.float32)
        u = jnp.dot(x, wu_ref[0], preferred_element_type=jnp.float32)
        o_ref[...] = _gelu(g) * u


def _gmm1(te, tot, xs, w_gate_up):
    grid_spec = pltpu.PrefetchScalarGridSpec(
        num_scalar_prefetch=2,
        grid=(FF // FN, G),
        in_specs=[
            pl.BlockSpec((TM, H), lambda n, m, te, tot: (m, 0)),
            pl.BlockSpec((1, H, FN), lambda n, m, te, tot: (te[m], 0, n)),
            pl.BlockSpec((1, H, FN), lambda n, m, te, tot: (te[m], 0, n + FF // FN)),
        ],
        out_specs=pl.BlockSpec((TM, FN), lambda n, m, te, tot: (m, n)),
    )
    return pl.pallas_call(
        _gmm1_body,
        grid_spec=grid_spec,
        out_shape=jax.ShapeDtypeStruct((XS, FF), jnp.float32),
    )(te, tot, xs, w_gate_up, w_gate_up)


def _gmm2_body(te_ref, tot_ref, a_ref, wd_ref, o_ref):
    m = pl.program_id(1)

    @pl.when(m < tot_ref[0])
    def _():
        o_ref[...] = jnp.dot(a_ref[...], wd_ref[0],
                             preferred_element_type=jnp.float32)


def _gmm2(te, tot, act, w_down):
    grid_spec = pltpu.PrefetchScalarGridSpec(
        num_scalar_prefetch=2,
        grid=(H // FN, G),
        in_specs=[
            pl.BlockSpec((TM, FF), lambda n, m, te, tot: (m, 0)),
            pl.BlockSpec((1, FF, FN), lambda n, m, te, tot: (te[m], 0, n)),
        ],
        out_specs=pl.BlockSpec((TM, FN), lambda n, m, te, tot: (m, n)),
    )
    return pl.pallas_call(
        _gmm2_body,
        grid_spec=grid_spec,
        out_shape=jax.ShapeDtypeStruct((XS, H), jnp.float32),
    )(te, tot, act, w_down)


def _fin_body(y0_ref, y1_ref, w0_ref, w1_ref, hid_ref, qm_ref, o_ref):
    moe = y0_ref[...] * w0_ref[...] + y1_ref[...] * w1_ref[...]
    o_ref[...] = hid_ref[...] + _rms(moe, qm_ref[...])


def _finalize(yp, w0, w1, hidden, qm):
    return pl.pallas_call(
        _fin_body,
        grid=(S // RT,),
        in_specs=[
            pl.BlockSpec((RT, H), lambda r: (r, 0)),
            pl.BlockSpec((RT, H), lambda r: (r + S // RT, 0)),
            pl.BlockSpec((RT, 1), lambda r: (r, 0)),
            pl.BlockSpec((RT, 1), lambda r: (r, 0)),
            pl.BlockSpec((RT, H), lambda r: (r, 0)),
            pl.BlockSpec((1, H), lambda r: (0, 0)),
        ],
        out_specs=pl.BlockSpec((RT, H), lambda r: (r, 0)),
        out_shape=jax.ShapeDtypeStruct((S, H), jnp.float32),
    )(yp, yp, w0, w1, hidden, qm)


def kernel(positions, hidden_states, w_qkv, w_o, gate_w, w_gate_up, w_down,
           pre_attn_norm_w, post_attn_norm_w, pre_moe_norm_w, post_moe_norm_w):
    hs = hidden_states.reshape(S, H)
    pos = positions.reshape(S).astype(jnp.float32)
    inv = 1.0 / (THETA ** (jnp.arange(0, HD, 2, dtype=jnp.float32) / HD))
    f = pos[:, None] * inv[None, :]
    cosf = jnp.concatenate([jnp.cos(f), jnp.cos(f)], axis=1)
    sinf = jnp.concatenate([jnp.sin(f), jnp.sin(f)], axis=1)

    qkv = _qkv(hs, pre_attn_norm_w.reshape(1, H), w_qkv, cosf, sinf)
    q = qkv[:, : NH * HD]
    k = qkv[:, NH * HD : (NH + NKV) * HD]
    v = qkv[:, (NH + NKV) * HD :]
    ao = _attn(q, k, v)
    hidden, xn, logits = _oproj(ao, w_o, hs, post_attn_norm_w.reshape(1, H),
                                pre_moe_norm_w.reshape(1, H), gate_w)
    p0, p1, w0, w1, te, tot = _route(logits)
    p0f = p0.reshape(S)
    p1f = p1.reshape(S)
    xs = _dispatch_sc(xn, p0f, p1f)
    act = _gmm1(te.reshape(TM), tot.reshape(1), xs, w_gate_up)
    ys = _gmm2(te.reshape(TM), tot.reshape(1), act, w_down)
    yp = _combine_sc(ys, p0f, p1f)
    out = _finalize(yp, w0, w1, hidden, post_moe_norm_w.reshape(1, H))
    return out.reshape(B, S, H)
